# baseline (speedup 1.0000x reference)
import jax
import jax.numpy as jnp
from jax.experimental import pallas as pl
from jax.experimental.pallas import tpu as pltpu

E = 8
TOP = 2
EMB = 384
HID = 2 * EMB
NB = 16
HH = 32
WW = 32
HW = HH * WW
NG = 4


def _fold_kernel(W1_ref, W2_ref, W3_ref, b1_ref, b2_ref, s_ref, t_ref,
                 A_ref, c_ref, W3b_ref):
    W12 = jnp.dot(W2_ref[0], W1_ref[0], preferred_element_type=jnp.float32)
    A_ref[0] = (W12 * s_ref[0]).astype(jnp.bfloat16)
    b12 = jnp.dot(W2_ref[0], b1_ref[0], preferred_element_type=jnp.float32)
    c_ref[0] = s_ref[0] * (b12 + b2_ref[0]) + t_ref[0]
    W3b_ref[0] = W3_ref[0].astype(jnp.bfloat16)


def _gate_kernel(x_ref, g_ref, i0_ref, i1_ref, w0_ref, w1_ref, xb_ref):
    xb_ref[...] = x_ref[...].astype(jnp.bfloat16)
    x0 = jnp.mean(x_ref[...], axis=2)
    for g in range(NG):
        logits = jnp.dot(x0, g_ref[g], preferred_element_type=jnp.float32)
        m = jnp.max(logits, axis=1, keepdims=True)
        ex = jnp.exp(logits - m)
        p = ex / jnp.sum(ex, axis=1, keepdims=True)
        ii = jax.lax.broadcasted_iota(jnp.int32, (NB, E), 1)
        m0 = jnp.max(p, axis=1, keepdims=True)
        i0 = jnp.min(jnp.where(p >= m0, ii, E), axis=1)
        p2 = jnp.where(ii == i0[:, None], -jnp.inf, p)
        m1 = jnp.max(p2, axis=1, keepdims=True)
        i1 = jnp.min(jnp.where(p2 >= m1, ii, E), axis=1)
        eb = jnp.exp(m1[:, 0] - m0[:, 0])
        w0 = 1.0 / (1.0 + eb)
        i0_ref[g] = i0
        i1_ref[g] = i1
        w0_ref[g] = w0
        w1_ref[g] = 1.0 - w0


def _apply_kernel(idx_ref, w_ref, x_ref, A0_ref, A1_ref, c0_ref, c1_ref,
                  W30_ref, W31_ref, b30_ref, b31_ref,
                  o0_ref, o1_ref, o2_ref, o3_ref):
    b = pl.program_id(0)
    g = pl.program_id(1)
    s0 = (g * NB + b) * TOP
    w0 = w_ref[s0]
    w1 = w_ref[s0 + 1]
    X = x_ref[0]
    H0 = (w0 * jnp.maximum(
        jnp.dot(A0_ref[0], X, preferred_element_type=jnp.float32) + c0_ref[0], 0.0
    )).astype(jnp.bfloat16)
    H1 = (w1 * jnp.maximum(
        jnp.dot(A1_ref[0], X, preferred_element_type=jnp.float32) + c1_ref[0], 0.0
    )).astype(jnp.bfloat16)
    Y = (jnp.dot(W30_ref[0], H0, preferred_element_type=jnp.float32)
         + jnp.dot(W31_ref[0], H1, preferred_element_type=jnp.float32)
         + (w0 * b30_ref[0] + w1 * b31_ref[0]))

    for gi, o_ref in enumerate((o0_ref, o1_ref, o2_ref, o3_ref)):
        @pl.when(g == gi)
        def _(o_ref=o_ref):
            o_ref[0] = Y


def _expert_spec(block, k):
    return pl.BlockSpec(block, lambda b, g, idx, w: (idx[(g * NB + b) * TOP + k], 0, 0))


def kernel(x, gates, W1, b1, W2, b2, gamma, beta, rm, rv, W3, b3):
    x3 = x.reshape(NB, EMB, HW)
    s = gamma * jax.lax.rsqrt(rv + 1e-5)
    t = beta - rm * s
    s_c = s.reshape(E, HID, 1)
    t_c = t.reshape(E, HID, 1)
    b1_c = b1.reshape(E, HID, 1)
    b2_c = b2.reshape(E, HID, 1)
    b3_c = b3.reshape(E, EMB, 1)

    A, c, W3b = pl.pallas_call(
        _fold_kernel,
        grid=(E,),
        in_specs=[
            pl.BlockSpec((1, HID, EMB), lambda e: (e, 0, 0)),
            pl.BlockSpec((1, HID, HID), lambda e: (e, 0, 0)),
            pl.BlockSpec((1, EMB, HID), lambda e: (e, 0, 0)),
            pl.BlockSpec((1, HID, 1), lambda e: (e, 0, 0)),
            pl.BlockSpec((1, HID, 1), lambda e: (e, 0, 0)),
            pl.BlockSpec((1, HID, 1), lambda e: (e, 0, 0)),
            pl.BlockSpec((1, HID, 1), lambda e: (e, 0, 0)),
        ],
        out_specs=[
            pl.BlockSpec((1, HID, EMB), lambda e: (e, 0, 0)),
            pl.BlockSpec((1, HID, 1), lambda e: (e, 0, 0)),
            pl.BlockSpec((1, EMB, HID), lambda e: (e, 0, 0)),
        ],
        out_shape=[
            jax.ShapeDtypeStruct((E, HID, EMB), jnp.bfloat16),
            jax.ShapeDtypeStruct((E, HID, 1), jnp.float32),
            jax.ShapeDtypeStruct((E, EMB, HID), jnp.bfloat16),
        ],
    )(W1, W2, W3, b1_c, b2_c, s_c, t_c)

    i0, i1, w0, w1, xb = pl.pallas_call(
        _gate_kernel,
        out_shape=[
            jax.ShapeDtypeStruct((NG, NB), jnp.int32),
            jax.ShapeDtypeStruct((NG, NB), jnp.int32),
            jax.ShapeDtypeStruct((NG, NB), jnp.float32),
            jax.ShapeDtypeStruct((NG, NB), jnp.float32),
            jax.ShapeDtypeStruct((NB, EMB, HW), jnp.bfloat16),
        ],
    )(x3, gates)

    flat_idx = jnp.stack([i0, i1], axis=-1).reshape(-1)
    wts = jnp.stack([w0, w1], axis=-1).reshape(-1)

    outs = pl.pallas_call(
        _apply_kernel,
        grid_spec=pltpu.PrefetchScalarGridSpec(
            num_scalar_prefetch=2,
            grid=(NB, NG),
            in_specs=[
                pl.BlockSpec((1, EMB, HW), lambda b, g, idx, w: (b, 0, 0)),
                _expert_spec((1, HID, EMB), 0),
                _expert_spec((1, HID, EMB), 1),
                _expert_spec((1, HID, 1), 0),
                _expert_spec((1, HID, 1), 1),
                _expert_spec((1, EMB, HID), 0),
                _expert_spec((1, EMB, HID), 1),
                _expert_spec((1, EMB, 1), 0),
                _expert_spec((1, EMB, 1), 1),
            ],
            out_specs=[
                pl.BlockSpec((1, EMB, HW), lambda b, g, idx, w: (b, 0, 0))
                for _ in range(NG)
            ],
        ),
        out_shape=[
            jax.ShapeDtypeStruct((NB, EMB, HW), jnp.float32) for _ in range(NG)
        ],
    )(flat_idx, wts, xb, A, A, c, c, W3b, W3b, b3_c, b3_c)

    return tuple(o.reshape(NB, EMB, HH, WW) for o in outs)


# speedup vs baseline: 1.0755x; 1.0755x over previous
import jax
import jax.numpy as jnp
from jax.experimental import pallas as pl
from jax.experimental.pallas import tpu as pltpu

E = 8
TOP = 2
EMB = 384
HID = 2 * EMB
NB = 16
HH = 32
WW = 32
HW = HH * WW
NG = 4


def _fold_kernel(W1_ref, W2_ref, W3_ref, b1_ref, b2_ref, s_ref, t_ref,
                 A_ref, c_ref, W3b_ref):
    W12 = jnp.dot(W2_ref[0], W1_ref[0], preferred_element_type=jnp.float32)
    A_ref[0] = (W12 * s_ref[0]).astype(jnp.bfloat16)
    b12 = jnp.dot(W2_ref[0], b1_ref[0], preferred_element_type=jnp.float32)
    c_ref[0] = s_ref[0] * (b12 + b2_ref[0]) + t_ref[0]
    W3b_ref[0] = W3_ref[0].astype(jnp.bfloat16)


def _gate_kernel(x_ref, g_ref, i0_ref, i1_ref, w0_ref, w1_ref, xb_ref):
    xb_ref[...] = x_ref[...].astype(jnp.bfloat16)
    x0 = jnp.mean(x_ref[...], axis=2)
    for g in range(NG):
        logits = jnp.dot(x0, g_ref[g], preferred_element_type=jnp.float32)
        m = jnp.max(logits, axis=1, keepdims=True)
        ex = jnp.exp(logits - m)
        p = ex / jnp.sum(ex, axis=1, keepdims=True)
        ii = jax.lax.broadcasted_iota(jnp.int32, (NB, E), 1)
        m0 = jnp.max(p, axis=1, keepdims=True)
        i0 = jnp.min(jnp.where(p >= m0, ii, E), axis=1)
        p2 = jnp.where(ii == i0[:, None], -jnp.inf, p)
        m1 = jnp.max(p2, axis=1, keepdims=True)
        i1 = jnp.min(jnp.where(p2 >= m1, ii, E), axis=1)
        eb = jnp.exp(m1[:, 0] - m0[:, 0])
        w0 = 1.0 / (1.0 + eb)
        i0_ref[g] = i0
        i1_ref[g] = i1
        w0_ref[g] = w0
        w1_ref[g] = 1.0 - w0


def _apply_kernel(idx_ref, w_ref, x_ref, *refs):
    b = pl.program_id(0)
    wrefs = refs[: NG * TOP * 4]
    orefs = refs[NG * TOP * 4:]
    X = x_ref[0]
    for g in range(NG):
        Yg = None
        for k in range(TOP):
            i = (g * TOP + k) * 4
            A_ref, c_ref, W3_ref, b3_ref = wrefs[i], wrefs[i + 1], wrefs[i + 2], wrefs[i + 3]
            w = w_ref[(g * NB + b) * TOP + k]
            H = jnp.maximum(
                jnp.dot(A_ref[0], X, preferred_element_type=jnp.float32) + c_ref[0],
                0.0,
            ).astype(jnp.bfloat16)
            W3w = (W3_ref[0] * w.astype(jnp.bfloat16)).astype(jnp.bfloat16)
            Yk = jnp.dot(W3w, H, preferred_element_type=jnp.float32) + w * b3_ref[0]
            Yg = Yk if Yg is None else Yg + Yk
        orefs[g][0] = Yg


def _expert_spec(block, g, k):
    return pl.BlockSpec(block, lambda b, idx, w: (idx[(g * NB + b) * TOP + k], 0, 0))


def kernel(x, gates, W1, b1, W2, b2, gamma, beta, rm, rv, W3, b3):
    x3 = x.reshape(NB, EMB, HW)
    s = gamma * jax.lax.rsqrt(rv + 1e-5)
    t = beta - rm * s
    s_c = s.reshape(E, HID, 1)
    t_c = t.reshape(E, HID, 1)
    b1_c = b1.reshape(E, HID, 1)
    b2_c = b2.reshape(E, HID, 1)
    b3_c = b3.reshape(E, EMB, 1)

    A, c, W3b = pl.pallas_call(
        _fold_kernel,
        grid=(E,),
        in_specs=[
            pl.BlockSpec((1, HID, EMB), lambda e: (e, 0, 0)),
            pl.BlockSpec((1, HID, HID), lambda e: (e, 0, 0)),
            pl.BlockSpec((1, EMB, HID), lambda e: (e, 0, 0)),
            pl.BlockSpec((1, HID, 1), lambda e: (e, 0, 0)),
            pl.BlockSpec((1, HID, 1), lambda e: (e, 0, 0)),
            pl.BlockSpec((1, HID, 1), lambda e: (e, 0, 0)),
            pl.BlockSpec((1, HID, 1), lambda e: (e, 0, 0)),
        ],
        out_specs=[
            pl.BlockSpec((1, HID, EMB), lambda e: (e, 0, 0)),
            pl.BlockSpec((1, HID, 1), lambda e: (e, 0, 0)),
            pl.BlockSpec((1, EMB, HID), lambda e: (e, 0, 0)),
        ],
        out_shape=[
            jax.ShapeDtypeStruct((E, HID, EMB), jnp.bfloat16),
            jax.ShapeDtypeStruct((E, HID, 1), jnp.float32),
            jax.ShapeDtypeStruct((E, EMB, HID), jnp.bfloat16),
        ],
    )(W1, W2, W3, b1_c, b2_c, s_c, t_c)

    i0, i1, w0, w1, xb = pl.pallas_call(
        _gate_kernel,
        out_shape=[
            jax.ShapeDtypeStruct((NG, NB), jnp.int32),
            jax.ShapeDtypeStruct((NG, NB), jnp.int32),
            jax.ShapeDtypeStruct((NG, NB), jnp.float32),
            jax.ShapeDtypeStruct((NG, NB), jnp.float32),
            jax.ShapeDtypeStruct((NB, EMB, HW), jnp.bfloat16),
        ],
    )(x3, gates)

    flat_idx = jnp.stack([i0, i1], axis=-1).reshape(-1)
    wts = jnp.stack([w0, w1], axis=-1).reshape(-1)

    in_specs = [pl.BlockSpec((1, EMB, HW), lambda b, idx, w: (b, 0, 0))]
    args = [xb]
    for g in range(NG):
        for k in range(TOP):
            in_specs += [
                _expert_spec((1, HID, EMB), g, k),
                _expert_spec((1, HID, 1), g, k),
                _expert_spec((1, EMB, HID), g, k),
                _expert_spec((1, EMB, 1), g, k),
            ]
            args += [A, c, W3b, b3_c]

    outs = pl.pallas_call(
        _apply_kernel,
        grid_spec=pltpu.PrefetchScalarGridSpec(
            num_scalar_prefetch=2,
            grid=(NB,),
            in_specs=in_specs,
            out_specs=[
                pl.BlockSpec((1, EMB, HW), lambda b, idx, w: (b, 0, 0))
                for _ in range(NG)
            ],
        ),
        out_shape=[
            jax.ShapeDtypeStruct((NB, EMB, HW), jnp.float32) for _ in range(NG)
        ],
        compiler_params=pltpu.CompilerParams(vmem_limit_bytes=110 * 1024 * 1024),
    )(flat_idx, wts, *args)

    return tuple(o.reshape(NB, EMB, HH, WW) for o in outs)


# speedup vs baseline: 1.1715x; 1.0893x over previous
import jax
import jax.numpy as jnp
from jax.experimental import pallas as pl
from jax.experimental.pallas import tpu as pltpu

E = 8
TOP = 2
EMB = 384
HID = 2 * EMB
NB = 16
HH = 32
WW = 32
HW = HH * WW
NG = 4


def _fold_kernel(W1_ref, W2_ref, W3_ref, v_ref, A_ref, c_ref, W3b_ref):
    b1 = v_ref[0][:, 0:1]
    b2 = v_ref[0][:, 1:2]
    s = v_ref[0][:, 2:3]
    t = v_ref[0][:, 3:4]
    W12 = jnp.dot(W2_ref[0], W1_ref[0], preferred_element_type=jnp.float32)
    A_ref[0] = (W12 * s).astype(jnp.bfloat16)
    b12 = jnp.dot(W2_ref[0], b1, preferred_element_type=jnp.float32)
    c_ref[0] = s * (b12 + b2) + t
    W3b_ref[0] = W3_ref[0].astype(jnp.bfloat16)


def _gate_kernel(x_ref, g_ref, i0_ref, i1_ref, w0_ref, w1_ref):
    x0 = jnp.mean(x_ref[...], axis=2)
    for g in range(NG):
        logits = jnp.dot(x0, g_ref[g], preferred_element_type=jnp.float32)
        m = jnp.max(logits, axis=1, keepdims=True)
        ex = jnp.exp(logits - m)
        p = ex / jnp.sum(ex, axis=1, keepdims=True)
        ii = jax.lax.broadcasted_iota(jnp.int32, (NB, E), 1)
        m0 = jnp.max(p, axis=1, keepdims=True)
        i0 = jnp.min(jnp.where(p >= m0, ii, E), axis=1)
        p2 = jnp.where(ii == i0[:, None], -jnp.inf, p)
        m1 = jnp.max(p2, axis=1, keepdims=True)
        i1 = jnp.min(jnp.where(p2 >= m1, ii, E), axis=1)
        eb = jnp.exp(m1[:, 0] - m0[:, 0])
        w0 = 1.0 / (1.0 + eb)
        i0_ref[g] = i0
        i1_ref[g] = i1
        w0_ref[g] = w0
        w1_ref[g] = 1.0 - w0


def _apply_kernel(i0_ref, i1_ref, w0_ref, w1_ref, x_ref, *refs):
    b = pl.program_id(0)
    wrefs = refs[: NG * TOP * 4]
    orefs = refs[NG * TOP * 4:]
    X = x_ref[0].astype(jnp.bfloat16)
    for g in range(NG):
        Yg = None
        for k in range(TOP):
            i = (g * TOP + k) * 4
            A_ref, c_ref, W3_ref, b3_ref = wrefs[i], wrefs[i + 1], wrefs[i + 2], wrefs[i + 3]
            w = (w0_ref if k == 0 else w1_ref)[g, b]
            H = jnp.maximum(
                jnp.dot(A_ref[0], X, preferred_element_type=jnp.float32) + c_ref[0],
                0.0,
            ).astype(jnp.bfloat16)
            W3w = (W3_ref[0] * w.astype(jnp.bfloat16)).astype(jnp.bfloat16)
            Yk = jnp.dot(W3w, H, preferred_element_type=jnp.float32) + w * b3_ref[0]
            Yg = Yk if Yg is None else Yg + Yk
        orefs[g][0] = Yg.astype(jnp.bfloat16)


def _expert_spec(block, g, k):
    def imap(b, i0, i1, w0, w1, g=g, k=k):
        iref = i0 if k == 0 else i1
        return (iref[g, b], 0, 0)
    return pl.BlockSpec(block, imap)


def kernel(x, gates, W1, b1, W2, b2, gamma, beta, rm, rv, W3, b3):
    x3 = x.reshape(NB, EMB, HW)
    s = gamma * jax.lax.rsqrt(rv + 1e-5)
    t = beta - rm * s
    vecs = jnp.stack([b1, b2, s, t], axis=-1)
    b3_c = b3.reshape(E, EMB, 1)

    A, c, W3b = pl.pallas_call(
        _fold_kernel,
        grid=(E,),
        in_specs=[
            pl.BlockSpec((1, HID, EMB), lambda e: (e, 0, 0)),
            pl.BlockSpec((1, HID, HID), lambda e: (e, 0, 0)),
            pl.BlockSpec((1, EMB, HID), lambda e: (e, 0, 0)),
            pl.BlockSpec((1, HID, 4), lambda e: (e, 0, 0)),
        ],
        out_specs=[
            pl.BlockSpec((1, HID, EMB), lambda e: (e, 0, 0)),
            pl.BlockSpec((1, HID, 1), lambda e: (e, 0, 0)),
            pl.BlockSpec((1, EMB, HID), lambda e: (e, 0, 0)),
        ],
        out_shape=[
            jax.ShapeDtypeStruct((E, HID, EMB), jnp.bfloat16),
            jax.ShapeDtypeStruct((E, HID, 1), jnp.float32),
            jax.ShapeDtypeStruct((E, EMB, HID), jnp.bfloat16),
        ],
    )(W1, W2, W3, vecs)

    i0, i1, w0, w1 = pl.pallas_call(
        _gate_kernel,
        out_shape=[
            jax.ShapeDtypeStruct((NG, NB), jnp.int32),
            jax.ShapeDtypeStruct((NG, NB), jnp.int32),
            jax.ShapeDtypeStruct((NG, NB), jnp.float32),
            jax.ShapeDtypeStruct((NG, NB), jnp.float32),
        ],
    )(x3, gates)

    in_specs = [pl.BlockSpec((1, EMB, HW), lambda b, i0, i1, w0, w1: (b, 0, 0))]
    args = [x3]
    for g in range(NG):
        for k in range(TOP):
            in_specs += [
                _expert_spec((1, HID, EMB), g, k),
                _expert_spec((1, HID, 1), g, k),
                _expert_spec((1, EMB, HID), g, k),
                _expert_spec((1, EMB, 1), g, k),
            ]
            args += [A, c, W3b, b3_c]

    outs = pl.pallas_call(
        _apply_kernel,
        grid_spec=pltpu.PrefetchScalarGridSpec(
            num_scalar_prefetch=4,
            grid=(NB,),
            in_specs=in_specs,
            out_specs=[
                pl.BlockSpec((1, EMB, HW), lambda b, i0, i1, w0, w1: (b, 0, 0))
                for _ in range(NG)
            ],
        ),
        out_shape=[
            jax.ShapeDtypeStruct((NB, EMB, HW), jnp.bfloat16) for _ in range(NG)
        ],
        compiler_params=pltpu.CompilerParams(vmem_limit_bytes=110 * 1024 * 1024),
    )(i0, i1, w0, w1, *args)

    return tuple(
        o.astype(jnp.float32).reshape(NB, EMB, HH, WW) for o in outs
    )
